# baseline (speedup 1.0000x reference)
import functools

import jax
import jax.numpy as jnp
from jax import lax
from jax.experimental import pallas as pl
from jax.experimental.pallas import tpu as pltpu
from jax.experimental.pallas import tpu_sc as plsc

N = 10000
E = 160000
NPAD = 10240
EPAD = 163840
G = 64
DIM = 512
NC = 2
NT = 16
K = 128
BN = 1024
EPS = 1e-5


def _sc_mesh():
    return plsc.VectorSubcoreMesh(
        core_axis_name="c", subcore_axis_name="s", num_cores=NC,
        num_subcores=NT)


def _make_agg(C):
    EPT = EPAD // NT
    NB = EPT // K
    RPT = NPAD // NT
    npass = C // NC

    def body(*refs):
        hs = refs[0:C]
        src3 = refs[C]
        dst3 = refs[C + 1]
        zeros = refs[C + 2]
        outs = refs[C + 3:C + 3 + C]
        scr = refs[C + 3 + C:]
        idx_s = scr[0]
        idx_d = scr[1:5]
        rows = scr[5:7]
        accum = scr[7]
        si = scr[8:12]
        sg = scr[12:14]
        ss = scr[14:16]
        cid = lax.axis_index("c")
        sid = lax.axis_index("s")
        r0 = pl.multiple_of(sid * RPT, RPT)
        pltpu.sync_copy(src3.at[sid], idx_s)

        def fire_idx(i, sl):
            pltpu.async_copy(dst3.at[sid, i], idx_d[sl], si[sl])

        def drain_idx(sl):
            pltpu.make_async_copy(dst3.at[sid, 0], idx_d[sl], si[sl]).wait()

        for p in range(npass):
            for cv in range(NC):
                ch = p * NC + cv

                @pl.when(cid == cv)
                def _(ch=ch):
                    pltpu.sync_copy(zeros.at[pl.ds(r0, RPT)],
                                    accum.at[pl.ds(r0, RPT)])
                    plsc.subcore_barrier()
                    fire_idx(0, 0)
                    fire_idx(1, 1)

                    def bstep(i, carry):
                        sl = lax.rem(i, 4)
                        for slv in range(4):

                            @pl.when(sl == slv)
                            def _(bv=slv % 2, slv=slv, slv2=(slv + 2) % 4):
                                drain_idx(slv)

                                @pl.when(i >= 2)
                                def _():
                                    pltpu.make_async_copy(
                                        rows[bv], accum.at[idx_d[slv2]],
                                        ss[bv]).wait()
                                pltpu.async_copy(
                                    hs[ch].at[idx_s.at[i]], rows[bv],
                                    sg[bv])

                                @pl.when(i + 2 < NB)
                                def _():
                                    fire_idx(i + 2, slv2)
                                pltpu.make_async_copy(
                                    hs[ch].at[idx_s.at[0]], rows[bv],
                                    sg[bv]).wait()
                                pltpu.async_copy(
                                    rows[bv], accum.at[idx_d[slv]],
                                    ss[bv], add=True)
                        return carry

                    lax.fori_loop(0, NB, bstep, 0)
                    for i in (NB - 2, NB - 1):
                        pltpu.make_async_copy(
                            rows[i % 2], accum.at[idx_d[i % 4]],
                            ss[i % 2]).wait()
                    plsc.subcore_barrier()
                    pltpu.sync_copy(accum.at[pl.ds(r0, RPT)],
                                    outs[ch].at[pl.ds(r0, RPT)])

    out_type = [jax.ShapeDtypeStruct((NPAD, 128), jnp.float32)
                for _ in range(C)]
    scratch = (
        [pltpu.VMEM((NB, K), jnp.int32)]
        + [pltpu.VMEM((K,), jnp.int32) for _ in range(4)]
        + [pltpu.VMEM((K, 128), jnp.float32) for _ in range(2)]
        + [pltpu.VMEM_SHARED((NPAD, 128), jnp.float32)]
        + [pltpu.SemaphoreType.DMA for _ in range(8)]
    )
    return pl.kernel(body, out_type=out_type, mesh=_sc_mesh(),
                     scratch_types=scratch)


def _mlp(hs, aggs, W1, b1, W2, b2):
    C = len(hs)
    NBLK = NPAD // BN

    def kern(*refs):
        h_refs = refs[0:C]
        a_refs = refs[C:2 * C]
        w1, b1r, w2, b2r = refs[2 * C:2 * C + 4]
        z_outs = refs[2 * C + 4:2 * C + 8]
        stats = refs[2 * C + 8]
        i = pl.program_id(0)
        u = jnp.zeros((BN, DIM), dtype=jnp.float32)
        for c in range(C):
            xc = (h_refs[c][...] + a_refs[c][...]).astype(jnp.bfloat16)
            u = u + jnp.dot(xc, w1[c * 128:(c + 1) * 128, :],
                            preferred_element_type=jnp.float32)
        u = jnp.maximum(u + b1r[...], 0.0)
        z = jnp.dot(u.astype(jnp.bfloat16), w2[...],
                    preferred_element_type=jnp.float32) + b2r[...]
        z = jnp.maximum(z, 0.0)
        for c in range(4):
            z_outs[c][...] = z[:, c * 128:(c + 1) * 128]
        row = i * BN + lax.broadcasted_iota(jnp.int32, (BN, 1), 0)
        zm = jnp.where(row < N, z, 0.0)
        s1 = jnp.sum(zm, axis=0, keepdims=True)
        s2 = jnp.sum(zm * zm, axis=0, keepdims=True)
        contrib = jnp.concatenate(
            [s1, s2, jnp.zeros((6, DIM), jnp.float32)], axis=0)
        prev = jnp.where(i == 0, jnp.zeros_like(contrib), stats[...])
        stats[...] = prev + contrib

    row_spec = pl.BlockSpec((BN, 128), lambda i: (i, 0))
    full = lambda shape: pl.BlockSpec(shape, lambda i: (0, 0))
    in_specs = ([row_spec] * (2 * C)
                + [full(W1.shape), full((1, DIM)), full(W2.shape),
                   full((1, DIM))])
    out_specs = [row_spec] * 4 + [full((8, DIM))]
    out_shape = ([jax.ShapeDtypeStruct((NPAD, 128), jnp.float32)] * 4
                 + [jax.ShapeDtypeStruct((8, DIM), jnp.float32)])
    outs = pl.pallas_call(
        kern,
        grid=(NBLK,),
        in_specs=in_specs,
        out_specs=out_specs,
        out_shape=out_shape,
        compiler_params=pltpu.CompilerParams(
            dimension_semantics=("arbitrary",)),
    )(*hs, *aggs, W1.astype(jnp.bfloat16), b1.reshape(1, DIM),
      W2.astype(jnp.bfloat16), b2.reshape(1, DIM))
    return outs[:4], outs[4]


def _normalize(zs, stats, gamma, beta):
    NBLK = NPAD // BN

    def kern(*refs):
        z_refs = refs[0:4]
        st, gr, br = refs[4:7]
        outs = refs[7:11]
        i = pl.program_id(0)
        mean = st[0:1, :] * (1.0 / N)
        var = st[1:2, :] * (1.0 / N) - mean * mean
        scale = gr[...] * lax.rsqrt(var + EPS)
        shift = br[...] - mean * scale
        row = i * BN + lax.broadcasted_iota(jnp.int32, (BN, 1), 0)
        valid = row < N
        for c in range(4):
            outs[c][...] = jnp.where(
                valid,
                z_refs[c][...] * scale[:, c * 128:(c + 1) * 128]
                + shift[:, c * 128:(c + 1) * 128],
                0.0)

    row_spec = pl.BlockSpec((BN, 128), lambda i: (i, 0))
    full = lambda shape: pl.BlockSpec(shape, lambda i: (0, 0))
    outs = pl.pallas_call(
        kern,
        grid=(NBLK,),
        in_specs=[row_spec] * 4 + [full((8, DIM)), full((1, DIM)),
                                   full((1, DIM))],
        out_specs=[row_spec] * 4,
        out_shape=[jax.ShapeDtypeStruct((NPAD, 128), jnp.float32)] * 4,
        compiler_params=pltpu.CompilerParams(
            dimension_semantics=("arbitrary",)),
    )(*zs, stats, gamma.reshape(1, DIM), beta.reshape(1, DIM))
    return outs


def _make_pool(NCH):
    RPT = NPAD // NT
    NB = RPT // K

    def body(*refs):
        chunks = refs[0:NCH]
        bids = refs[NCH]
        zeros = refs[NCH + 1]
        out = refs[NCH + 2]
        idx_b, rows, accum, sem = refs[NCH + 3:]
        cid = lax.axis_index("c")
        sid = lax.axis_index("s")
        for j in range(NCH):
            cv = j % NC

            @pl.when(cid == cv)
            def _(j=j):
                @pl.when(sid == 0)
                def _():
                    pltpu.sync_copy(zeros, accum)
                plsc.subcore_barrier()

                def bstep(i, carry):
                    r0 = pl.multiple_of(sid * RPT + i * K, K)
                    pltpu.sync_copy(bids.at[pl.ds(r0, K)], idx_b)
                    pltpu.sync_copy(chunks[j].at[pl.ds(r0, K)], rows)
                    pltpu.sync_copy(rows, accum.at[idx_b], add=True)
                    return carry

                lax.fori_loop(0, NB, bstep, 0)
                plsc.subcore_barrier()

                @pl.when(sid == 0)
                def _():
                    pltpu.sync_copy(accum.at[pl.ds(0, G)], out.at[j])

    out_type = jax.ShapeDtypeStruct((NCH, G, 128), jnp.float32)
    scratch = [
        pltpu.VMEM((K,), jnp.int32),
        pltpu.VMEM((K, 128), jnp.float32),
        pltpu.VMEM_SHARED((G + 8, 128), jnp.float32),
        pltpu.SemaphoreType.DMA,
    ]
    return pl.kernel(body, out_type=out_type, mesh=_sc_mesh(),
                     scratch_types=scratch)


def kernel(x, edge_index, batch,
           W1_0, b1_0, W2_0, b2_0, gamma_0, beta_0,
           W1_1, b1_1, W2_1, b2_1, gamma_1, beta_1,
           W1_2, b1_2, W2_2, b2_2, gamma_2, beta_2):
    params = [
        (W1_0, b1_0, W2_0, b2_0, gamma_0, beta_0),
        (W1_1, b1_1, W2_1, b2_1, gamma_1, beta_1),
        (W1_2, b1_2, W2_2, b2_2, gamma_2, beta_2),
    ]
    epad = EPAD - E
    src = jnp.concatenate(
        [edge_index[0],
         N + jnp.arange(epad, dtype=jnp.int32) % (NPAD - N)]).reshape(
             NT, -1, K)
    dst = jnp.concatenate(
        [edge_index[1],
         (jnp.arange(epad, dtype=jnp.int32) * 97) % NPAD]).reshape(NT, -1, K)
    bids = jnp.concatenate(
        [batch, jnp.full((NPAD - N,), G, dtype=jnp.int32)])
    xp = jnp.pad(x, ((0, NPAD - N), (0, 0)))
    zeros_big = jnp.zeros((NPAD, 128), jnp.float32)
    zeros_pool = jnp.zeros((G + 8, 128), jnp.float32)

    F_IN = x.shape[1]
    hs = [xp[:, c * 128:(c + 1) * 128] for c in range(F_IN // 128)]

    agg2 = _make_agg(2)
    agg4 = _make_agg(4)
    norm_chunks = []
    for li, (W1, b1, W2, b2, g, be) in enumerate(params):
        aggfn = agg2 if len(hs) == 2 else agg4
        aggs = aggfn(*hs, src, dst, zeros_big)
        if not isinstance(aggs, (list, tuple)):
            aggs = (aggs,)
        zs, stats = _mlp(hs, aggs, W1, b1, W2, b2)
        hs = _normalize(zs, stats, g, be)
        norm_chunks.extend(hs)

    pooled = _make_pool(12)(*norm_chunks, bids, zeros_pool)
    xpool = pooled.transpose(1, 0, 2).reshape(G, 12 * 128)
    xs = jnp.concatenate([c[:N] for c in norm_chunks], axis=1)
    return xpool, xs


# speedup vs baseline: 1.0774x; 1.0774x over previous
import functools

import jax
import jax.numpy as jnp
from jax import lax
from jax.experimental import pallas as pl
from jax.experimental.pallas import tpu as pltpu
from jax.experimental.pallas import tpu_sc as plsc

N = 10000
E = 160000
NPAD = 10240
EPAD = 163840
G = 64
DIM = 512
NC = 2
NT = 16
K = 128
BN = 1024
EPS = 1e-5


def _sc_mesh():
    return plsc.VectorSubcoreMesh(
        core_axis_name="c", subcore_axis_name="s", num_cores=NC,
        num_subcores=NT)


def _make_agg(C):
    EPT = EPAD // NT
    NB = EPT // K
    RPT = NPAD // NT
    npass = C // NC

    def body(*refs):
        hs = refs[0:C]
        src3 = refs[C]
        dst3 = refs[C + 1]
        zeros = refs[C + 2]
        outs = refs[C + 3:C + 3 + C]
        scr = refs[C + 3 + C:]
        idx_s = scr[0]
        idx_d = scr[1:5]
        rows = scr[5:7]
        accum = scr[7]
        si = scr[8:12]
        sg = scr[12:14]
        ss = scr[14:16]
        cid = lax.axis_index("c")
        sid = lax.axis_index("s")
        r0 = pl.multiple_of(sid * RPT, RPT)
        pltpu.sync_copy(src3.at[sid], idx_s)

        def fire_idx(i, sl):
            pltpu.async_copy(dst3.at[sid, i], idx_d[sl], si[sl])

        def drain_idx(sl):
            pltpu.make_async_copy(dst3.at[sid, 0], idx_d[sl], si[sl]).wait()

        for p in range(npass):
            for cv in range(NC):
                ch = p * NC + cv

                @pl.when(cid == cv)
                def _(ch=ch):
                    pltpu.sync_copy(zeros.at[pl.ds(r0, RPT)],
                                    accum.at[pl.ds(r0, RPT)])
                    plsc.subcore_barrier()
                    fire_idx(0, 0)
                    fire_idx(1, 1)

                    def bstep(i, carry):
                        sl = lax.rem(i, 4)
                        for slv in range(4):

                            @pl.when(sl == slv)
                            def _(bv=slv % 2, slv=slv, slv2=(slv + 2) % 4):
                                drain_idx(slv)

                                @pl.when(i >= 2)
                                def _():
                                    pltpu.make_async_copy(
                                        rows[bv], accum.at[idx_d[slv2]],
                                        ss[bv]).wait()
                                pltpu.async_copy(
                                    hs[ch].at[idx_s.at[i]], rows[bv],
                                    sg[bv])

                                @pl.when(i + 2 < NB)
                                def _():
                                    fire_idx(i + 2, slv2)
                                pltpu.make_async_copy(
                                    hs[ch].at[idx_s.at[0]], rows[bv],
                                    sg[bv]).wait()
                                pltpu.async_copy(
                                    rows[bv], accum.at[idx_d[slv]],
                                    ss[bv], add=True)
                        return carry

                    lax.fori_loop(0, NB, bstep, 0)
                    for i in (NB - 2, NB - 1):
                        pltpu.make_async_copy(
                            rows[i % 2], accum.at[idx_d[i % 4]],
                            ss[i % 2]).wait()
                    plsc.subcore_barrier()
                    pltpu.sync_copy(accum.at[pl.ds(r0, RPT)],
                                    outs[ch].at[pl.ds(r0, RPT)])

    out_type = [jax.ShapeDtypeStruct((NPAD, 128), jnp.float32)
                for _ in range(C)]
    scratch = (
        [pltpu.VMEM((NB, K), jnp.int32)]
        + [pltpu.VMEM((K,), jnp.int32) for _ in range(4)]
        + [pltpu.VMEM((K, 128), jnp.float32) for _ in range(2)]
        + [pltpu.VMEM_SHARED((NPAD, 128), jnp.float32)]
        + [pltpu.SemaphoreType.DMA for _ in range(8)]
    )
    return pl.kernel(body, out_type=out_type, mesh=_sc_mesh(),
                     scratch_types=scratch)


def _mlp(hs, aggs, W1, b1, W2, b2):
    C = len(hs)
    NBLK = NPAD // BN

    def kern(*refs):
        h_refs = refs[0:C]
        a_refs = refs[C:2 * C]
        w1, b1r, w2, b2r = refs[2 * C:2 * C + 4]
        z_outs = refs[2 * C + 4:2 * C + 8]
        stats = refs[2 * C + 8]
        i = pl.program_id(0)
        u = jnp.zeros((BN, DIM), dtype=jnp.float32)
        for c in range(C):
            xc = h_refs[c][...] + a_refs[c][...]
            u = u + jnp.dot(xc, w1[c * 128:(c + 1) * 128, :],
                            preferred_element_type=jnp.float32)
        u = jnp.maximum(u + b1r[...], 0.0)
        z = jnp.dot(u, w2[...], preferred_element_type=jnp.float32) + b2r[...]
        z = jnp.maximum(z, 0.0)
        for c in range(4):
            z_outs[c][...] = z[:, c * 128:(c + 1) * 128]
        row = i * BN + lax.broadcasted_iota(jnp.int32, (BN, 1), 0)
        zm = jnp.where(row < N, z, 0.0)
        s1 = jnp.sum(zm, axis=0, keepdims=True)
        s2 = jnp.sum(zm * zm, axis=0, keepdims=True)
        contrib = jnp.concatenate(
            [s1, s2, jnp.zeros((6, DIM), jnp.float32)], axis=0)
        prev = jnp.where(i == 0, jnp.zeros_like(contrib), stats[...])
        stats[...] = prev + contrib

    row_spec = pl.BlockSpec((BN, 128), lambda i: (i, 0))
    full = lambda shape: pl.BlockSpec(shape, lambda i: (0, 0))
    in_specs = ([row_spec] * (2 * C)
                + [full(W1.shape), full((1, DIM)), full(W2.shape),
                   full((1, DIM))])
    out_specs = [row_spec] * 4 + [full((8, DIM))]
    out_shape = ([jax.ShapeDtypeStruct((NPAD, 128), jnp.float32)] * 4
                 + [jax.ShapeDtypeStruct((8, DIM), jnp.float32)])
    outs = pl.pallas_call(
        kern,
        grid=(NBLK,),
        in_specs=in_specs,
        out_specs=out_specs,
        out_shape=out_shape,
        compiler_params=pltpu.CompilerParams(
            dimension_semantics=("arbitrary",)),
    )(*hs, *aggs, W1, b1.reshape(1, DIM), W2, b2.reshape(1, DIM))
    return outs[:4], outs[4]


def _normalize(zs, stats, gamma, beta, bids3):
    NBLK = NPAD // BN

    def kern(*refs):
        z_refs = refs[0:4]
        st, gr, br, bid_ref = refs[4:8]
        outs = refs[8:12]
        pool = refs[12]
        i = pl.program_id(0)
        mean = st[0:1, :] * (1.0 / N)
        var = st[1:2, :] * (1.0 / N) - mean * mean
        scale = gr[...] * lax.rsqrt(var + EPS)
        shift = br[...] - mean * scale
        row = i * BN + lax.broadcasted_iota(jnp.int32, (BN, 1), 0)
        valid = row < N
        hn = []
        for c in range(4):
            hc = jnp.where(
                valid,
                z_refs[c][...] * scale[:, c * 128:(c + 1) * 128]
                + shift[:, c * 128:(c + 1) * 128],
                0.0)
            outs[c][...] = hc
            hn.append(hc)
        bid = bid_ref[0]
        onehot = jnp.where(
            lax.broadcasted_iota(jnp.int32, (G, BN), 0) == bid,
            1.0, 0.0)
        contrib = jnp.dot(onehot, jnp.concatenate(hn, axis=1),
                          preferred_element_type=jnp.float32,
                          precision=lax.Precision.HIGHEST)
        pool[...] = jnp.where(i == 0, contrib, pool[...] + contrib)

    row_spec = pl.BlockSpec((BN, 128), lambda i: (i, 0))
    full = lambda shape: pl.BlockSpec(shape, lambda i: (0, 0))
    outs = pl.pallas_call(
        kern,
        grid=(NBLK,),
        in_specs=[row_spec] * 4 + [full((8, DIM)), full((1, DIM)),
                                   full((1, DIM)),
                                   pl.BlockSpec((1, 1, BN),
                                                lambda i: (i, 0, 0))],
        out_specs=[row_spec] * 4 + [full((G, DIM))],
        out_shape=([jax.ShapeDtypeStruct((NPAD, 128), jnp.float32)] * 4
                   + [jax.ShapeDtypeStruct((G, DIM), jnp.float32)]),
        compiler_params=pltpu.CompilerParams(
            dimension_semantics=("arbitrary",)),
    )(*zs, stats, gamma.reshape(1, DIM), beta.reshape(1, DIM), bids3)
    return outs[:4], outs[4]


def kernel(x, edge_index, batch,
           W1_0, b1_0, W2_0, b2_0, gamma_0, beta_0,
           W1_1, b1_1, W2_1, b2_1, gamma_1, beta_1,
           W1_2, b1_2, W2_2, b2_2, gamma_2, beta_2):
    params = [
        (W1_0, b1_0, W2_0, b2_0, gamma_0, beta_0),
        (W1_1, b1_1, W2_1, b2_1, gamma_1, beta_1),
        (W1_2, b1_2, W2_2, b2_2, gamma_2, beta_2),
    ]
    epad = EPAD - E
    src = jnp.concatenate(
        [edge_index[0],
         N + jnp.arange(epad, dtype=jnp.int32) % (NPAD - N)]).reshape(
             NT, -1, K)
    dst = jnp.concatenate(
        [edge_index[1],
         (jnp.arange(epad, dtype=jnp.int32) * 97) % NPAD]).reshape(NT, -1, K)
    bids3 = jnp.concatenate(
        [batch, jnp.full((NPAD - N,), G, dtype=jnp.int32)]).reshape(
            NPAD // BN, 1, BN)
    xp = jnp.pad(x, ((0, NPAD - N), (0, 0)))
    zeros_big = jnp.zeros((NPAD, 128), jnp.float32)

    F_IN = x.shape[1]
    hs = [xp[:, c * 128:(c + 1) * 128] for c in range(F_IN // 128)]

    agg2 = _make_agg(2)
    agg4 = _make_agg(4)
    norm_chunks = []
    pools = []
    for li, (W1, b1, W2, b2, g, be) in enumerate(params):
        aggfn = agg2 if len(hs) == 2 else agg4
        aggs = aggfn(*hs, src, dst, zeros_big)
        if not isinstance(aggs, (list, tuple)):
            aggs = (aggs,)
        zs, stats = _mlp(hs, aggs, W1, b1, W2, b2)
        hs, pool_l = _normalize(zs, stats, g, be, bids3)
        norm_chunks.extend(hs)
        pools.append(pool_l)

    xpool = jnp.concatenate(pools, axis=1)
    xs = jnp.concatenate([c[:N] for c in norm_chunks], axis=1)
    return xpool, xs


# speedup vs baseline: 1.2960x; 1.2029x over previous
import functools

import jax
import jax.numpy as jnp
from jax import lax
from jax.experimental import pallas as pl
from jax.experimental.pallas import tpu as pltpu
from jax.experimental.pallas import tpu_sc as plsc

N = 10000
E = 160000
NPAD = 10240
EPAD = 163840
G = 64
DIM = 512
NC = 2
NT = 16
K = 80
BN = 1024
EPS = 1e-5


def _sc_mesh():
    return plsc.VectorSubcoreMesh(
        core_axis_name="c", subcore_axis_name="s", num_cores=NC,
        num_subcores=NT)


def _make_agg(C):
    EPT = EPAD // NT
    NB = EPT // K
    RPT = NPAD // NT
    npass = C // NC
    NBUF = 3
    NSL = 6

    def body(*refs):
        hs = refs[0:C]
        src3 = refs[C]
        dst3 = refs[C + 1]
        zeros = refs[C + 2]
        outs = refs[C + 3:C + 3 + C]
        scr = refs[C + 3 + C:]
        idx_s = scr[0]
        idx_d = scr[1:1 + NSL]
        rows = scr[1 + NSL:1 + NSL + NBUF]
        accum = scr[1 + NSL + NBUF]
        si = scr[2 + NSL + NBUF:2 + 2 * NSL + NBUF]
        sg = scr[2 + 2 * NSL + NBUF:2 + 2 * NSL + 2 * NBUF]
        ss = scr[2 + 2 * NSL + 2 * NBUF:]
        cid = lax.axis_index("c")
        sid = lax.axis_index("s")
        r0 = pl.multiple_of(sid * RPT, RPT)
        pltpu.sync_copy(src3.at[sid], idx_s)

        def fire_idx(i, sl):
            pltpu.async_copy(dst3.at[sid, i], idx_d[sl], si[sl])

        def drain_idx(sl):
            pltpu.make_async_copy(dst3.at[sid, 0], idx_d[sl], si[sl]).wait()

        def fire_gather(h_ref, i, b):
            pltpu.async_copy(h_ref.at[idx_s.at[i]], rows[b], sg[b])

        def drain_gather(h_ref, b):
            pltpu.make_async_copy(h_ref.at[idx_s.at[0]], rows[b],
                                  sg[b]).wait()

        def fire_scatter(accum, b, sl):
            pltpu.async_copy(rows[b], accum.at[idx_d[sl]], ss[b], add=True)

        def drain_scatter(accum, b):
            pltpu.make_async_copy(rows[b], accum.at[idx_d[0]], ss[b]).wait()

        for p in range(npass):
            for cv in range(NC):
                ch = p * NC + cv

                @pl.when(cid == cv)
                def _(ch=ch):
                    pltpu.sync_copy(zeros.at[pl.ds(r0, RPT)],
                                    accum.at[pl.ds(r0, RPT)])
                    plsc.subcore_barrier()
                    for j in range(NBUF):
                        fire_idx(j, j)
                    drain_idx(0)
                    fire_gather(hs[ch], 0, 0)

                    def bstep(i, carry):
                        sl = lax.rem(i, NSL)
                        for slv in range(NSL):

                            @pl.when(sl == slv)
                            def _(bv=slv % NBUF, slv=slv):
                                @pl.when(i >= 2)
                                def _():
                                    drain_scatter(accum, (bv + 1) % NBUF)

                                @pl.when(i + 1 < NB)
                                def _():
                                    drain_idx((slv + 1) % NSL)
                                    fire_gather(hs[ch], i + 1,
                                                (bv + 1) % NBUF)

                                @pl.when(i + NBUF < NB)
                                def _():
                                    fire_idx(i + NBUF, (slv + NBUF) % NSL)
                                drain_gather(hs[ch], bv)
                                fire_scatter(accum, bv, slv)
                        return carry

                    lax.fori_loop(0, NB, bstep, 0)
                    for i in (NB - 2, NB - 1):
                        drain_scatter(accum, i % NBUF)
                    plsc.subcore_barrier()
                    pltpu.sync_copy(accum.at[pl.ds(r0, RPT)],
                                    outs[ch].at[pl.ds(r0, RPT)])

    out_type = [jax.ShapeDtypeStruct((NPAD, 128), jnp.float32)
                for _ in range(C)]
    scratch = (
        [pltpu.VMEM((NB, K), jnp.int32)]
        + [pltpu.VMEM((K,), jnp.int32) for _ in range(NSL)]
        + [pltpu.VMEM((K, 128), jnp.float32) for _ in range(NBUF)]
        + [pltpu.VMEM_SHARED((NPAD, 128), jnp.float32)]
        + [pltpu.SemaphoreType.DMA for _ in range(NSL + 2 * NBUF)]
    )
    return pl.kernel(body, out_type=out_type, mesh=_sc_mesh(),
                     scratch_types=scratch)


def _mlp(hs, aggs, W1, b1, W2, b2):
    C = len(hs)
    NBLK = NPAD // BN

    def kern(*refs):
        h_refs = refs[0:C]
        a_refs = refs[C:2 * C]
        w1, b1r, w2, b2r = refs[2 * C:2 * C + 4]
        z_outs = refs[2 * C + 4:2 * C + 8]
        stats = refs[2 * C + 8]
        i = pl.program_id(0)
        u = jnp.zeros((BN, DIM), dtype=jnp.float32)
        for c in range(C):
            xc = h_refs[c][...] + a_refs[c][...]
            u = u + jnp.dot(xc, w1[c * 128:(c + 1) * 128, :],
                            preferred_element_type=jnp.float32)
        u = jnp.maximum(u + b1r[...], 0.0)
        z = jnp.dot(u, w2[...], preferred_element_type=jnp.float32) + b2r[...]
        z = jnp.maximum(z, 0.0)
        for c in range(4):
            z_outs[c][...] = z[:, c * 128:(c + 1) * 128]
        row = i * BN + lax.broadcasted_iota(jnp.int32, (BN, 1), 0)
        zm = jnp.where(row < N, z, 0.0)
        s1 = jnp.sum(zm, axis=0, keepdims=True)
        s2 = jnp.sum(zm * zm, axis=0, keepdims=True)
        contrib = jnp.concatenate(
            [s1, s2, jnp.zeros((6, DIM), jnp.float32)], axis=0)
        prev = jnp.where(i == 0, jnp.zeros_like(contrib), stats[...])
        stats[...] = prev + contrib

    row_spec = pl.BlockSpec((BN, 128), lambda i: (i, 0))
    full = lambda shape: pl.BlockSpec(shape, lambda i: (0, 0))
    in_specs = ([row_spec] * (2 * C)
                + [full(W1.shape), full((1, DIM)), full(W2.shape),
                   full((1, DIM))])
    out_specs = [row_spec] * 4 + [full((8, DIM))]
    out_shape = ([jax.ShapeDtypeStruct((NPAD, 128), jnp.float32)] * 4
                 + [jax.ShapeDtypeStruct((8, DIM), jnp.float32)])
    outs = pl.pallas_call(
        kern,
        grid=(NBLK,),
        in_specs=in_specs,
        out_specs=out_specs,
        out_shape=out_shape,
        compiler_params=pltpu.CompilerParams(
            dimension_semantics=("arbitrary",)),
    )(*hs, *aggs, W1, b1.reshape(1, DIM), W2, b2.reshape(1, DIM))
    return outs[:4], outs[4]


def _normalize(zs, stats, gamma, beta, bids3):
    NBLK = NPAD // BN

    def kern(*refs):
        z_refs = refs[0:4]
        st, gr, br, bid_ref = refs[4:8]
        outs = refs[8:12]
        pool = refs[12]
        i = pl.program_id(0)
        mean = st[0:1, :] * (1.0 / N)
        var = st[1:2, :] * (1.0 / N) - mean * mean
        scale = gr[...] * lax.rsqrt(var + EPS)
        shift = br[...] - mean * scale
        row = i * BN + lax.broadcasted_iota(jnp.int32, (BN, 1), 0)
        valid = row < N
        hn = []
        for c in range(4):
            hc = jnp.where(
                valid,
                z_refs[c][...] * scale[:, c * 128:(c + 1) * 128]
                + shift[:, c * 128:(c + 1) * 128],
                0.0)
            outs[c][...] = hc
            hn.append(hc)
        bid = bid_ref[0]
        onehot = jnp.where(
            lax.broadcasted_iota(jnp.int32, (G, BN), 0) == bid,
            1.0, 0.0)
        contrib = jnp.dot(onehot, jnp.concatenate(hn, axis=1),
                          preferred_element_type=jnp.float32,
                          precision=lax.Precision.HIGHEST)
        pool[...] = jnp.where(i == 0, contrib, pool[...] + contrib)

    row_spec = pl.BlockSpec((BN, 128), lambda i: (i, 0))
    full = lambda shape: pl.BlockSpec(shape, lambda i: (0, 0))
    outs = pl.pallas_call(
        kern,
        grid=(NBLK,),
        in_specs=[row_spec] * 4 + [full((8, DIM)), full((1, DIM)),
                                   full((1, DIM)),
                                   pl.BlockSpec((1, 1, BN),
                                                lambda i: (i, 0, 0))],
        out_specs=[row_spec] * 4 + [full((G, DIM))],
        out_shape=([jax.ShapeDtypeStruct((NPAD, 128), jnp.float32)] * 4
                   + [jax.ShapeDtypeStruct((G, DIM), jnp.float32)]),
        compiler_params=pltpu.CompilerParams(
            dimension_semantics=("arbitrary",)),
    )(*zs, stats, gamma.reshape(1, DIM), beta.reshape(1, DIM), bids3)
    return outs[:4], outs[4]


def kernel(x, edge_index, batch,
           W1_0, b1_0, W2_0, b2_0, gamma_0, beta_0,
           W1_1, b1_1, W2_1, b2_1, gamma_1, beta_1,
           W1_2, b1_2, W2_2, b2_2, gamma_2, beta_2):
    params = [
        (W1_0, b1_0, W2_0, b2_0, gamma_0, beta_0),
        (W1_1, b1_1, W2_1, b2_1, gamma_1, beta_1),
        (W1_2, b1_2, W2_2, b2_2, gamma_2, beta_2),
    ]
    epad = EPAD - E
    src = jnp.concatenate(
        [edge_index[0],
         N + jnp.arange(epad, dtype=jnp.int32) % (NPAD - N)]).reshape(
             NT, -1, K)
    dst = jnp.concatenate(
        [edge_index[1],
         (jnp.arange(epad, dtype=jnp.int32) * 97) % NPAD]).reshape(NT, -1, K)
    bids3 = jnp.concatenate(
        [batch, jnp.full((NPAD - N,), G, dtype=jnp.int32)]).reshape(
            NPAD // BN, 1, BN)
    xp = jnp.pad(x, ((0, NPAD - N), (0, 0)))
    zeros_big = jnp.zeros((NPAD, 128), jnp.float32)

    F_IN = x.shape[1]
    hs = [xp[:, c * 128:(c + 1) * 128] for c in range(F_IN // 128)]

    agg2 = _make_agg(2)
    agg4 = _make_agg(4)
    norm_chunks = []
    pools = []
    for li, (W1, b1, W2, b2, g, be) in enumerate(params):
        aggfn = agg2 if len(hs) == 2 else agg4
        aggs = aggfn(*hs, src, dst, zeros_big)
        if not isinstance(aggs, (list, tuple)):
            aggs = (aggs,)
        zs, stats = _mlp(hs, aggs, W1, b1, W2, b2)
        hs, pool_l = _normalize(zs, stats, g, be, bids3)
        norm_chunks.extend(hs)
        pools.append(pool_l)

    xpool = jnp.concatenate(pools, axis=1)
    xs = jnp.concatenate([c[:N] for c in norm_chunks], axis=1)
    return xpool, xs


# speedup vs baseline: 1.3295x; 1.0258x over previous
import functools

import jax
import jax.numpy as jnp
from jax import lax
from jax.experimental import pallas as pl
from jax.experimental.pallas import tpu as pltpu
from jax.experimental.pallas import tpu_sc as plsc

N = 10000
E = 160000
NPAD = 10240
EPAD = 163840
G = 64
DIM = 512
NC = 2
NT = 16
K = 80
BN = 1024
EPS = 1e-5


def _sc_mesh():
    return plsc.VectorSubcoreMesh(
        core_axis_name="c", subcore_axis_name="s", num_cores=NC,
        num_subcores=NT)


def _make_agg(C):
    EPT = EPAD // NT
    NB = EPT // K
    RPT = NPAD // NT
    npass = C // NC
    NBUF = 3
    NSL = 6

    def body(*refs):
        hs = refs[0:C]
        src3 = refs[C]
        dst3 = refs[C + 1]
        zeros = refs[C + 2]
        outs = refs[C + 3:C + 3 + C]
        scr = refs[C + 3 + C:]
        idx_s = scr[0]
        idx_d = scr[1:1 + NSL]
        rows = scr[1 + NSL:1 + NSL + NBUF]
        accum = scr[1 + NSL + NBUF]
        si = scr[2 + NSL + NBUF:2 + 2 * NSL + NBUF]
        sg = scr[2 + 2 * NSL + NBUF:2 + 2 * NSL + 2 * NBUF]
        ss = scr[2 + 2 * NSL + 2 * NBUF:]
        cid = lax.axis_index("c")
        sid = lax.axis_index("s")
        r0 = pl.multiple_of(sid * RPT, RPT)
        pltpu.sync_copy(src3.at[sid], idx_s)

        def fire_idx(i, sl):
            pltpu.async_copy(dst3.at[sid, i], idx_d[sl], si[sl])

        def drain_idx(sl):
            pltpu.make_async_copy(dst3.at[sid, 0], idx_d[sl], si[sl]).wait()

        def fire_gather(h_ref, i, b):
            pltpu.async_copy(h_ref.at[idx_s.at[i]], rows[b], sg[b])

        def drain_gather(h_ref, b):
            pltpu.make_async_copy(h_ref.at[idx_s.at[0]], rows[b],
                                  sg[b]).wait()

        def fire_scatter(accum, b, sl):
            pltpu.async_copy(rows[b], accum.at[idx_d[sl]], ss[b], add=True)

        def drain_scatter(accum, b):
            pltpu.make_async_copy(rows[b], accum.at[idx_d[0]], ss[b]).wait()

        for p in range(npass):
            for cv in range(NC):
                ch = p * NC + cv

                @pl.when(cid == cv)
                def _(ch=ch):
                    pltpu.sync_copy(zeros.at[pl.ds(r0, RPT)],
                                    accum.at[pl.ds(r0, RPT)])
                    plsc.subcore_barrier()
                    for j in range(NBUF):
                        fire_idx(j, j)
                    drain_idx(0)
                    fire_gather(hs[ch], 0, 0)

                    def bstep(i, carry):
                        sl = lax.rem(i, NSL)
                        for slv in range(NSL):

                            @pl.when(sl == slv)
                            def _(bv=slv % NBUF, slv=slv):
                                @pl.when(i >= NBUF - 1)
                                def _():
                                    drain_scatter(accum, (bv + 1) % NBUF)

                                @pl.when(i + 1 < NB)
                                def _():
                                    drain_idx((slv + 1) % NSL)
                                    fire_gather(hs[ch], i + 1,
                                                (bv + 1) % NBUF)

                                @pl.when(i + NBUF < NB)
                                def _():
                                    fire_idx(i + NBUF, (slv + NBUF) % NSL)
                                drain_gather(hs[ch], bv)
                                fire_scatter(accum, bv, slv)
                        return carry

                    lax.fori_loop(0, NB, bstep, 0)
                    for i in range(NB - NBUF + 1, NB):
                        drain_scatter(accum, i % NBUF)
                    plsc.subcore_barrier()
                    pltpu.sync_copy(accum.at[pl.ds(r0, RPT)],
                                    outs[ch].at[pl.ds(r0, RPT)])

    out_type = [jax.ShapeDtypeStruct((NPAD, 128), jnp.float32)
                for _ in range(C)]
    scratch = (
        [pltpu.VMEM((NB, K), jnp.int32)]
        + [pltpu.VMEM((K,), jnp.int32) for _ in range(NSL)]
        + [pltpu.VMEM((K, 128), jnp.float32) for _ in range(NBUF)]
        + [pltpu.VMEM_SHARED((NPAD, 128), jnp.float32)]
        + [pltpu.SemaphoreType.DMA for _ in range(NSL + 2 * NBUF)]
    )
    return pl.kernel(body, out_type=out_type, mesh=_sc_mesh(),
                     scratch_types=scratch)


def _mlp(hs, aggs, W1, b1, W2, b2):
    C = len(hs)
    NBLK = NPAD // BN

    def kern(*refs):
        h_refs = refs[0:C]
        a_refs = refs[C:2 * C]
        w1, b1r, w2, b2r = refs[2 * C:2 * C + 4]
        z_outs = refs[2 * C + 4:2 * C + 8]
        stats = refs[2 * C + 8]
        i = pl.program_id(0)
        u = jnp.zeros((BN, DIM), dtype=jnp.float32)
        for c in range(C):
            xc = h_refs[c][...] + a_refs[c][...]
            u = u + jnp.dot(xc, w1[c * 128:(c + 1) * 128, :],
                            preferred_element_type=jnp.float32)
        u = jnp.maximum(u + b1r[...], 0.0)
        z = jnp.dot(u, w2[...], preferred_element_type=jnp.float32) + b2r[...]
        z = jnp.maximum(z, 0.0)
        for c in range(4):
            z_outs[c][...] = z[:, c * 128:(c + 1) * 128]
        row = i * BN + lax.broadcasted_iota(jnp.int32, (BN, 1), 0)
        zm = jnp.where(row < N, z, 0.0)
        s1 = jnp.sum(zm, axis=0, keepdims=True)
        s2 = jnp.sum(zm * zm, axis=0, keepdims=True)
        contrib = jnp.concatenate(
            [s1, s2, jnp.zeros((6, DIM), jnp.float32)], axis=0)
        prev = jnp.where(i == 0, jnp.zeros_like(contrib), stats[...])
        stats[...] = prev + contrib

    row_spec = pl.BlockSpec((BN, 128), lambda i: (i, 0))
    full = lambda shape: pl.BlockSpec(shape, lambda i: (0, 0))
    in_specs = ([row_spec] * (2 * C)
                + [full(W1.shape), full((1, DIM)), full(W2.shape),
                   full((1, DIM))])
    out_specs = [row_spec] * 4 + [full((8, DIM))]
    out_shape = ([jax.ShapeDtypeStruct((NPAD, 128), jnp.float32)] * 4
                 + [jax.ShapeDtypeStruct((8, DIM), jnp.float32)])
    outs = pl.pallas_call(
        kern,
        grid=(NBLK,),
        in_specs=in_specs,
        out_specs=out_specs,
        out_shape=out_shape,
        compiler_params=pltpu.CompilerParams(
            dimension_semantics=("arbitrary",)),
    )(*hs, *aggs, W1, b1.reshape(1, DIM), W2, b2.reshape(1, DIM))
    return outs[:4], outs[4]


def _normalize(zs, stats, gamma, beta, bids3, col, slab, last):
    NBLK = NPAD // BN
    nch = 0 if last else 4

    def kern(*refs):
        z_refs = refs[0:4]
        st, gr, br, bid_ref = refs[4:8]
        outs = refs[9 if slab is not None else 8:]
        i = pl.program_id(0)
        mean = st[0:1, :] * (1.0 / N)
        var = st[1:2, :] * (1.0 / N) - mean * mean
        scale = gr[...] * lax.rsqrt(var + EPS)
        shift = br[...] - mean * scale
        row = i * BN + lax.broadcasted_iota(jnp.int32, (BN, 1), 0)
        valid = row < N
        hn = []
        for c in range(4):
            hc = jnp.where(
                valid,
                z_refs[c][...] * scale[:, c * 128:(c + 1) * 128]
                + shift[:, c * 128:(c + 1) * 128],
                0.0)
            if not last:
                outs[c][...] = hc
            hn.append(hc)
        hcat = jnp.concatenate(hn, axis=1)
        outs[nch + 1][...] = hcat
        bid = bid_ref[0]
        onehot = jnp.where(
            lax.broadcasted_iota(jnp.int32, (G, BN), 0) == bid,
            1.0, 0.0)
        contrib = jnp.dot(onehot, hcat,
                          preferred_element_type=jnp.float32,
                          precision=lax.Precision.HIGHEST)
        pool = outs[nch]
        pool[...] = jnp.where(i == 0, contrib, pool[...] + contrib)

    row_spec = pl.BlockSpec((BN, 128), lambda i: (i, 0))
    full = lambda shape: pl.BlockSpec(shape, lambda i: (0, 0))
    in_specs = [row_spec] * 4 + [full((8, DIM)), full((1, DIM)),
                                 full((1, DIM)),
                                 pl.BlockSpec((1, 1, BN),
                                              lambda i: (i, 0, 0))]
    args = [*zs, stats, gamma.reshape(1, DIM), beta.reshape(1, DIM), bids3]
    aliases = {}
    if slab is not None:
        in_specs.append(pl.BlockSpec(memory_space=pltpu.MemorySpace.HBM))
        args.append(slab)
        aliases = {8: nch + 1}
    out_specs = ([row_spec] * nch + [full((G, DIM))]
                 + [pl.BlockSpec((BN, DIM), lambda i: (i, col))])
    out_shape = ([jax.ShapeDtypeStruct((NPAD, 128), jnp.float32)] * nch
                 + [jax.ShapeDtypeStruct((G, DIM), jnp.float32)]
                 + [jax.ShapeDtypeStruct((N, 3 * DIM), jnp.float32)])
    outs = pl.pallas_call(
        kern,
        grid=(NBLK,),
        in_specs=in_specs,
        out_specs=out_specs,
        out_shape=out_shape,
        input_output_aliases=aliases,
        compiler_params=pltpu.CompilerParams(
            dimension_semantics=("arbitrary",)),
    )(*args)
    return outs[:nch], outs[nch], outs[nch + 1]


def kernel(x, edge_index, batch,
           W1_0, b1_0, W2_0, b2_0, gamma_0, beta_0,
           W1_1, b1_1, W2_1, b2_1, gamma_1, beta_1,
           W1_2, b1_2, W2_2, b2_2, gamma_2, beta_2):
    params = [
        (W1_0, b1_0, W2_0, b2_0, gamma_0, beta_0),
        (W1_1, b1_1, W2_1, b2_1, gamma_1, beta_1),
        (W1_2, b1_2, W2_2, b2_2, gamma_2, beta_2),
    ]
    epad = EPAD - E
    src = jnp.concatenate(
        [edge_index[0],
         N + jnp.arange(epad, dtype=jnp.int32) % (NPAD - N)]).reshape(
             NT, -1, K)
    dst = jnp.concatenate(
        [edge_index[1],
         (jnp.arange(epad, dtype=jnp.int32) * 97) % NPAD]).reshape(NT, -1, K)
    bids3 = jnp.concatenate(
        [batch, jnp.full((NPAD - N,), G, dtype=jnp.int32)]).reshape(
            NPAD // BN, 1, BN)
    xp = jnp.pad(x, ((0, NPAD - N), (0, 0)))
    zeros_big = jnp.zeros((NPAD, 128), jnp.float32)

    F_IN = x.shape[1]
    hs = [xp[:, c * 128:(c + 1) * 128] for c in range(F_IN // 128)]

    agg2 = _make_agg(2)
    agg4 = _make_agg(4)
    pools = []
    slab = None
    for li, (W1, b1, W2, b2, g, be) in enumerate(params):
        aggfn = agg2 if len(hs) == 2 else agg4
        aggs = aggfn(*hs, src, dst, zeros_big)
        if not isinstance(aggs, (list, tuple)):
            aggs = (aggs,)
        zs, stats = _mlp(hs, aggs, W1, b1, W2, b2)
        hs, pool_l, slab = _normalize(zs, stats, g, be, bids3, li, slab,
                                      last=(li == 2))
        pools.append(pool_l)

    xpool = jnp.concatenate(pools, axis=1)
    return xpool, slab
